# baseline (speedup 1.0000x reference)
import functools

import jax
import jax.numpy as jnp
from jax import lax
from jax.experimental import pallas as pl
from jax.experimental.pallas import tpu as pltpu
from jax.experimental.pallas import tpu_sc as plsc

_D = 64
_B = 16384
_V = 100000
_NC = 2
_NS = 16
_NW = _NC * _NS
_BPW = _B // _NW
_PPW = _BPW // 2
_CH = 128
_NCH = _BPW // _CH
_TBLK = 512


def _k1_body(x_ref, o_ref):
    eye = jnp.eye(_D, dtype=jnp.float32)
    o_ref[:, 0:_D] = jax.lax.dot_general(
        x_ref[...], eye, (((0,), (0,)), ((), ())),
        preferred_element_type=jnp.float32)


_k1 = pl.pallas_call(
    _k1_body,
    grid=((_V + _TBLK - 1) // _TBLK,),
    in_specs=[pl.BlockSpec((_D, _TBLK), lambda j: (0, j))],
    out_specs=pl.BlockSpec((_TBLK, 2 * _D), lambda j: (j, 0)),
    out_shape=jax.ShapeDtypeStruct((_V, 2 * _D), jnp.float32),
)


def _k2_body(rep_hbm, idx_hbm, table_hbm, out_hbm,
             idx_v, rows_v, rep_v, acc_v, sem_g, sem_r):
    c = lax.axis_index("c")
    s = lax.axis_index("s")
    wid = s * _NC + c
    base = wid * _BPW

    pltpu.sync_copy(idx_hbm.at[pl.ds(base, _BPW)], idx_v)
    rep_cp = pltpu.async_copy(rep_hbm.at[pl.ds(wid * _PPW, _PPW)], rep_v,
                              sem_r)
    gathers = []
    for j in range(_NCH):
        gathers.append(pltpu.async_copy(
            table_hbm.at[idx_v.at[pl.ds(j * _CH, _CH)]],
            rows_v.at[pl.ds(j * _CH, _CH)], sem_g))
    rep_cp.wait()
    for g in gathers:
        g.wait()

    nk = _D // 16

    def body(j, accs):
        new = list(accs)
        for k in range(nk):
            e0 = rows_v[2 * j, pl.ds(k * 16, 16)]
            r0 = rep_v[j, pl.ds(k * 16, 16)]
            d0 = e0 - r0
            e1 = rows_v[2 * j + 1, pl.ds(k * 16, 16)]
            r1 = rep_v[j, pl.ds(_D + k * 16, 16)]
            d1 = e1 - r1
            new[k] = new[k] + d0 * d0 + d1 * d1
        return tuple(new)

    zero = jnp.zeros((16,), jnp.float32)
    accs = lax.fori_loop(0, _PPW, body, (zero,) * nk)
    total = accs[0]
    for a in accs[1:]:
        total = total + a
    acc_v[...] = total * (1.0 / (_B * _D))
    pltpu.sync_copy(acc_v, out_hbm.at[wid])


@functools.partial(
    pl.kernel,
    out_type=jax.ShapeDtypeStruct((_NW, 16), jnp.float32),
    mesh=plsc.VectorSubcoreMesh(core_axis_name="c", subcore_axis_name="s"),
    compiler_params=pltpu.CompilerParams(use_tc_tiling_on_sc=True),
    scratch_types=[
        pltpu.VMEM((_BPW,), jnp.int32),
        pltpu.VMEM((_BPW, 2 * _D), jnp.float32),
        pltpu.VMEM((_PPW, 2 * _D), jnp.float32),
        pltpu.VMEM((16,), jnp.float32),
        pltpu.SemaphoreType.DMA,
        pltpu.SemaphoreType.DMA,
    ],
)
def _k2(rep_hbm, idx_hbm, table_hbm, out_hbm,
        idx_v, rows_v, rep_v, acc_v, sem_g, sem_r):
    _k2_body(rep_hbm, idx_hbm, table_hbm, out_hbm,
             idx_v, rows_v, rep_v, acc_v, sem_g, sem_r)


def kernel(rep, expr, emb_weight):
    table128 = _k1(emb_weight.T)
    partials = _k2(rep.reshape(_B // 2, 2 * _D), expr.astype(jnp.int32),
                   table128)
    return jnp.sum(partials)


# speedup vs baseline: 2.1961x; 2.1961x over previous
import functools

import jax
import jax.numpy as jnp
from jax import lax
from jax.experimental import pallas as pl
from jax.experimental.pallas import tpu as pltpu
from jax.experimental.pallas import tpu_sc as plsc

_D = 64
_B = 16384
_V = 100000
_NC = 2
_NS = 16
_NW = _NC * _NS
_BPW = _B // _NW
_PPW = _BPW // 2
_CH = 128
_NCH = _BPW // _CH
_TBLK = 4096


def _k1_body(x_ref, o_ref):
    eye = jnp.eye(_D, dtype=jnp.float32)
    o_ref[:, 0:_D] = jax.lax.dot_general(
        x_ref[...], eye, (((0,), (0,)), ((), ())),
        preferred_element_type=jnp.float32)


_k1 = pl.pallas_call(
    _k1_body,
    grid=((_V + _TBLK - 1) // _TBLK,),
    in_specs=[pl.BlockSpec((_D, _TBLK), lambda j: (0, j))],
    out_specs=pl.BlockSpec((_TBLK, 2 * _D), lambda j: (j, 0)),
    out_shape=jax.ShapeDtypeStruct((_V, 2 * _D), jnp.float32),
)


def _k2_body(rep_hbm, idx_hbm, table_hbm, out_hbm,
             idx_v, rows_v, rep_v, acc_v, sem_g, sem_r):
    c = lax.axis_index("c")
    s = lax.axis_index("s")
    wid = s * _NC + c
    base = wid * _BPW

    pltpu.sync_copy(idx_hbm.at[pl.ds(base, _BPW)], idx_v)
    rep_cp = pltpu.async_copy(rep_hbm.at[pl.ds(wid * _PPW, _PPW)], rep_v,
                              sem_r)
    gathers = []
    for j in range(_NCH):
        gathers.append(pltpu.async_copy(
            table_hbm.at[idx_v.at[pl.ds(j * _CH, _CH)]],
            rows_v.at[pl.ds(j * _CH, _CH)], sem_g))
    rep_cp.wait()
    for g in gathers:
        g.wait()

    nk = _D // 16

    def body(j, accs):
        new = list(accs)
        for k in range(nk):
            e0 = rows_v[2 * j, pl.ds(k * 16, 16)]
            r0 = rep_v[j, pl.ds(k * 16, 16)]
            d0 = e0 - r0
            e1 = rows_v[2 * j + 1, pl.ds(k * 16, 16)]
            r1 = rep_v[j, pl.ds(_D + k * 16, 16)]
            d1 = e1 - r1
            new[k] = new[k] + d0 * d0 + d1 * d1
        return tuple(new)

    zero = jnp.zeros((16,), jnp.float32)
    accs = lax.fori_loop(0, _PPW, body, (zero,) * nk)
    total = accs[0]
    for a in accs[1:]:
        total = total + a
    acc_v[...] = total * (1.0 / (_B * _D))
    pltpu.sync_copy(acc_v, out_hbm.at[wid])


@functools.partial(
    pl.kernel,
    out_type=jax.ShapeDtypeStruct((_NW, 16), jnp.float32),
    mesh=plsc.VectorSubcoreMesh(core_axis_name="c", subcore_axis_name="s"),
    compiler_params=pltpu.CompilerParams(use_tc_tiling_on_sc=True),
    scratch_types=[
        pltpu.VMEM((_BPW,), jnp.int32),
        pltpu.VMEM((_BPW, 2 * _D), jnp.float32),
        pltpu.VMEM((_PPW, 2 * _D), jnp.float32),
        pltpu.VMEM((16,), jnp.float32),
        pltpu.SemaphoreType.DMA,
        pltpu.SemaphoreType.DMA,
    ],
)
def _k2(rep_hbm, idx_hbm, table_hbm, out_hbm,
        idx_v, rows_v, rep_v, acc_v, sem_g, sem_r):
    _k2_body(rep_hbm, idx_hbm, table_hbm, out_hbm,
             idx_v, rows_v, rep_v, acc_v, sem_g, sem_r)


def kernel(rep, expr, emb_weight):
    table128 = _k1(emb_weight.T)
    partials = _k2(rep.reshape(_B // 2, 2 * _D), expr.astype(jnp.int32),
                   table128)
    return jnp.sum(partials)


# speedup vs baseline: 2.4452x; 1.1134x over previous
import functools

import jax
import jax.numpy as jnp
from jax import lax
from jax.experimental import pallas as pl
from jax.experimental.pallas import tpu as pltpu
from jax.experimental.pallas import tpu_sc as plsc

_D = 64
_B = 16384
_V = 100000
_NC = 2
_NS = 16
_NW = _NC * _NS
_BPW = _B // _NW
_PPW = _BPW // 2
_CH = 128
_NCH = _BPW // _CH
_TBLK = 8192


def _k1_body(x_ref, o_ref):
    eye = jnp.eye(_D, dtype=jnp.float32)
    o_ref[:, 0:_D] = jax.lax.dot_general(
        x_ref[...], eye, (((0,), (0,)), ((), ())),
        preferred_element_type=jnp.float32)


_k1 = pl.pallas_call(
    _k1_body,
    grid=((_V + _TBLK - 1) // _TBLK,),
    in_specs=[pl.BlockSpec((_D, _TBLK), lambda j: (0, j))],
    out_specs=pl.BlockSpec((_TBLK, 2 * _D), lambda j: (j, 0)),
    out_shape=jax.ShapeDtypeStruct((_V, 2 * _D), jnp.float32),
)


def _k2_body(rep_hbm, idx_hbm, table_hbm, out_hbm,
             idx_v, rows_v, rep_v, acc_v, sem_g, sem_r):
    c = lax.axis_index("c")
    s = lax.axis_index("s")
    wid = s * _NC + c
    base = wid * _BPW

    pltpu.sync_copy(idx_hbm.at[pl.ds(base, _BPW)], idx_v)
    rep_cp = pltpu.async_copy(rep_hbm.at[pl.ds(wid * _PPW, _PPW)], rep_v,
                              sem_r)
    gathers = []
    for j in range(_NCH):
        gathers.append(pltpu.async_copy(
            table_hbm.at[idx_v.at[pl.ds(j * _CH, _CH)]],
            rows_v.at[pl.ds(j * _CH, _CH)], sem_g))
    rep_cp.wait()
    for g in gathers:
        g.wait()

    nk = _D // 16

    def body(j, accs):
        new = list(accs)
        for k in range(nk):
            e0 = rows_v[2 * j, pl.ds(k * 16, 16)]
            r0 = rep_v[j, pl.ds(k * 16, 16)]
            d0 = e0 - r0
            e1 = rows_v[2 * j + 1, pl.ds(k * 16, 16)]
            r1 = rep_v[j, pl.ds(_D + k * 16, 16)]
            d1 = e1 - r1
            new[k] = new[k] + d0 * d0 + d1 * d1
        return tuple(new)

    zero = jnp.zeros((16,), jnp.float32)
    accs = lax.fori_loop(0, _PPW, body, (zero,) * nk)
    total = accs[0]
    for a in accs[1:]:
        total = total + a
    acc_v[...] = total * (1.0 / (_B * _D))
    pltpu.sync_copy(acc_v, out_hbm.at[wid])


@functools.partial(
    pl.kernel,
    out_type=jax.ShapeDtypeStruct((_NW, 16), jnp.float32),
    mesh=plsc.VectorSubcoreMesh(core_axis_name="c", subcore_axis_name="s"),
    compiler_params=pltpu.CompilerParams(use_tc_tiling_on_sc=True),
    scratch_types=[
        pltpu.VMEM((_BPW,), jnp.int32),
        pltpu.VMEM((_BPW, 2 * _D), jnp.float32),
        pltpu.VMEM((_PPW, 2 * _D), jnp.float32),
        pltpu.VMEM((16,), jnp.float32),
        pltpu.SemaphoreType.DMA,
        pltpu.SemaphoreType.DMA,
    ],
)
def _k2(rep_hbm, idx_hbm, table_hbm, out_hbm,
        idx_v, rows_v, rep_v, acc_v, sem_g, sem_r):
    _k2_body(rep_hbm, idx_hbm, table_hbm, out_hbm,
             idx_v, rows_v, rep_v, acc_v, sem_g, sem_r)


def kernel(rep, expr, emb_weight):
    table128 = _k1(emb_weight.T)
    partials = _k2(rep.reshape(_B // 2, 2 * _D), expr.astype(jnp.int32),
                   table128)
    return jnp.sum(partials)


# speedup vs baseline: 2.5048x; 1.0244x over previous
import functools

import jax
import jax.numpy as jnp
from jax import lax
from jax.experimental import pallas as pl
from jax.experimental.pallas import tpu as pltpu
from jax.experimental.pallas import tpu_sc as plsc

_D = 64
_B = 16384
_V = 100000
_NC = 2
_NS = 16
_NW = _NC * _NS
_BPW = _B // _NW
_PPW = _BPW // 2
_CH = 128
_NCH = _BPW // _CH
_TBLK = 16384


def _k1_body(x_ref, o_ref):
    eye = jnp.eye(_D, dtype=jnp.float32)
    o_ref[:, 0:_D] = jax.lax.dot_general(
        x_ref[...], eye, (((0,), (0,)), ((), ())),
        preferred_element_type=jnp.float32)


_k1 = pl.pallas_call(
    _k1_body,
    grid=((_V + _TBLK - 1) // _TBLK,),
    in_specs=[pl.BlockSpec((_D, _TBLK), lambda j: (0, j))],
    out_specs=pl.BlockSpec((_TBLK, 2 * _D), lambda j: (j, 0)),
    out_shape=jax.ShapeDtypeStruct((_V, 2 * _D), jnp.float32),
)


def _k2_body(rep_hbm, idx_hbm, table_hbm, out_hbm,
             idx_v, rows_v, rep_v, acc_v, sem_g, sem_r):
    c = lax.axis_index("c")
    s = lax.axis_index("s")
    wid = s * _NC + c
    base = wid * _BPW

    pltpu.sync_copy(idx_hbm.at[pl.ds(base, _BPW)], idx_v)
    rep_cp = pltpu.async_copy(rep_hbm.at[pl.ds(wid * _PPW, _PPW)], rep_v,
                              sem_r)
    gathers = []
    for j in range(_NCH):
        gathers.append(pltpu.async_copy(
            table_hbm.at[idx_v.at[pl.ds(j * _CH, _CH)]],
            rows_v.at[pl.ds(j * _CH, _CH)], sem_g))
    rep_cp.wait()
    for g in gathers:
        g.wait()

    nk = _D // 16

    def body(j, accs):
        new = list(accs)
        for k in range(nk):
            e0 = rows_v[2 * j, pl.ds(k * 16, 16)]
            r0 = rep_v[j, pl.ds(k * 16, 16)]
            d0 = e0 - r0
            e1 = rows_v[2 * j + 1, pl.ds(k * 16, 16)]
            r1 = rep_v[j, pl.ds(_D + k * 16, 16)]
            d1 = e1 - r1
            new[k] = new[k] + d0 * d0 + d1 * d1
        return tuple(new)

    zero = jnp.zeros((16,), jnp.float32)
    accs = lax.fori_loop(0, _PPW, body, (zero,) * nk)
    total = accs[0]
    for a in accs[1:]:
        total = total + a
    acc_v[...] = total * (1.0 / (_B * _D))
    pltpu.sync_copy(acc_v, out_hbm.at[wid])


@functools.partial(
    pl.kernel,
    out_type=jax.ShapeDtypeStruct((_NW, 16), jnp.float32),
    mesh=plsc.VectorSubcoreMesh(core_axis_name="c", subcore_axis_name="s"),
    compiler_params=pltpu.CompilerParams(use_tc_tiling_on_sc=True),
    scratch_types=[
        pltpu.VMEM((_BPW,), jnp.int32),
        pltpu.VMEM((_BPW, 2 * _D), jnp.float32),
        pltpu.VMEM((_PPW, 2 * _D), jnp.float32),
        pltpu.VMEM((16,), jnp.float32),
        pltpu.SemaphoreType.DMA,
        pltpu.SemaphoreType.DMA,
    ],
)
def _k2(rep_hbm, idx_hbm, table_hbm, out_hbm,
        idx_v, rows_v, rep_v, acc_v, sem_g, sem_r):
    _k2_body(rep_hbm, idx_hbm, table_hbm, out_hbm,
             idx_v, rows_v, rep_v, acc_v, sem_g, sem_r)


def kernel(rep, expr, emb_weight):
    table128 = _k1(emb_weight.T)
    partials = _k2(rep.reshape(_B // 2, 2 * _D), expr.astype(jnp.int32),
                   table128)
    return jnp.sum(partials)
